# Initial kernel scaffold; baseline (speedup 1.0000x reference)
import jax
import jax.numpy as jnp
from jax.experimental import pallas as pl


def kernel(element_types, element_indices, style_vector, type_emb, idx_emb, W1, b1, W2, b2, pos_emb):
    raise NotImplementedError("write your pallas kernel here")


# speedup vs baseline: 1.0621x; 1.0621x over previous
import functools

import jax
import jax.numpy as jnp
from jax import lax
from jax.experimental import pallas as pl
from jax.experimental.pallas import tpu as pltpu
from jax.experimental.pallas import tpu_sc as plsc

B, S, D = 4096, 200, 256
NTYPE, NIDX = 5, 50
NC, NS = 2, 16
NW = NC * NS
NB = B // NW
LANES = 16


def _style_mask_body(types_ref, sv_ref, w1_ref, b1_ref, w2_ref, b2_ref,
                     styled_ref, mask_ref):
    h = jnp.dot(sv_ref[...], w1_ref[...], preferred_element_type=jnp.float32)
    h = jnp.maximum(h + b1_ref[...][None, :], 0.0)
    styled = jnp.dot(h, w2_ref[...], preferred_element_type=jnp.float32)
    styled_ref[...] = styled + b2_ref[...][None, :]
    mask_ref[...] = types_ref[...] == 0


def _tc_style_mask(types, style_vector, w1, b1, w2, b2):
    return pl.pallas_call(
        _style_mask_body,
        out_shape=[
            jax.ShapeDtypeStruct((B, D), jnp.float32),
            jax.ShapeDtypeStruct((B, S), jnp.bool_),
        ],
    )(types, style_vector, w1, b1, w2, b2)


@functools.partial(
    pl.kernel,
    out_type=jax.ShapeDtypeStruct((B, S, D), jnp.float32),
    mesh=plsc.VectorSubcoreMesh(
        core_axis_name="c", subcore_axis_name="s",
        num_cores=NC, num_subcores=NS),
    scratch_types=[
        pltpu.VMEM((NTYPE, D), jnp.float32),
        pltpu.VMEM((NIDX, D), jnp.float32),
        pltpu.VMEM((S, D), jnp.float32),
        pltpu.VMEM((S, D), jnp.float32),
        pltpu.VMEM((S,), jnp.int32),
        pltpu.VMEM((S,), jnp.int32),
        pltpu.VMEM((D,), jnp.float32),
    ],
)
def _sc_embed(types_hbm, inds_hbm, styled_hbm, temb_hbm, iemb_hbm, pemb_hbm,
              out_hbm, ttab, itab, ptab, outb, trow, irow, srow):
    wid = lax.axis_index("s") * NC + lax.axis_index("c")
    pltpu.sync_copy(temb_hbm, ttab)
    pltpu.sync_copy(iemb_hbm, itab)
    pltpu.sync_copy(pemb_hbm, ptab)

    def emit_pos(tvec, ivec, lane, s):
        t = tvec[lane]
        i = ivec[lane]
        for j in range(D // LANES):
            sl = pl.ds(j * LANES, LANES)
            outb[s, sl] = ttab[t, sl] + itab[i, sl] + ptab[s, sl]

        @pl.when(t == 1)
        def _():
            for j in range(D // LANES):
                sl = pl.ds(j * LANES, LANES)
                plsc.addupdate(outb.at[s, sl], srow[sl])

    def row_body(k, carry):
        b = wid * NB + k
        pltpu.sync_copy(types_hbm.at[b], trow)
        pltpu.sync_copy(inds_hbm.at[b], irow)
        pltpu.sync_copy(styled_hbm.at[b], srow)

        def chunk_body(c, carry2):
            s0 = c * LANES
            tvec = trow[pl.ds(s0, LANES)]
            ivec = irow[pl.ds(s0, LANES)]
            for lane in range(LANES):
                emit_pos(tvec, ivec, lane, s0 + lane)
            return carry2

        lax.fori_loop(0, S // LANES, chunk_body, 0)
        tvec = trow[pl.ds(S - LANES, LANES)]
        ivec = irow[pl.ds(S - LANES, LANES)]
        for lane in range(S % LANES, LANES):
            emit_pos(tvec, ivec, lane, S - LANES + lane)

        pltpu.sync_copy(outb, out_hbm.at[b])
        return carry

    lax.fori_loop(0, NB, row_body, 0)


def kernel(element_types, element_indices, style_vector, type_emb, idx_emb,
           W1, b1, W2, b2, pos_emb):
    types = element_types.astype(jnp.int32)
    inds = element_indices.astype(jnp.int32)
    styled, mask = _tc_style_mask(types, style_vector, W1, b1, W2, b2)
    final = _sc_embed(types, inds, styled, type_emb, idx_emb, pos_emb)
    return final, mask
